# Initial kernel scaffold; baseline (speedup 1.0000x reference)
import jax
import jax.numpy as jnp
from jax.experimental import pallas as pl


def kernel(x, Wq, bq, Wk, bk, Wg, bg, W1, b1, W2, b2, Wo, bo):
    raise NotImplementedError("write your pallas kernel here")


# speedup vs baseline: 1.5518x; 1.5518x over previous
import functools

import jax
import jax.numpy as jnp
from jax.experimental import pallas as pl
from jax.experimental.pallas import tpu as pltpu


def _moe_body(x_ref, wq_ref, bq_ref, wk_ref, bk_ref, wg_ref, bg_ref,
              w1_ref, b1_ref, w2_ref, b2_ref, wo_ref, bo_ref,
              out_ref, loss_ref, acc_ref, stat_ref,
              *, BT, D, H, E, HID, ODIM, NT, nb, ntok):
    h_id = pl.program_id(0)
    i = pl.program_id(1)
    bf = jnp.bfloat16

    @pl.when(jnp.logical_and(h_id == 0, i == 0))
    def _init_stats():
        stat_ref[...] = jnp.zeros_like(stat_ref)

    xb = x_ref[...]
    q = jnp.dot(xb, wq_ref[...], preferred_element_type=jnp.float32)
    q = q + bq_ref[0]
    g = jnp.dot(xb, wk_ref[...], preferred_element_type=jnp.float32)
    g = g + bk_ref[0]

    logits = jnp.dot(g.astype(bf), wg_ref[...],
                     preferred_element_type=jnp.float32) + bg_ref[...]
    mx = jnp.max(logits, axis=-1, keepdims=True)
    ex = jnp.exp(logits - mx)
    p = ex / jnp.sum(ex, axis=-1, keepdims=True)
    psum = jnp.sum(p, axis=0, keepdims=True)
    eq = logits >= mx
    lane = jax.lax.broadcasted_iota(jnp.int32, (BT, E), 1)
    am = jnp.min(jnp.where(eq, lane, E), axis=-1, keepdims=True)
    oh = (lane == am).astype(jnp.float32)
    csum = jnp.sum(oh, axis=0, keepdims=True)

    qb = q.astype(bf)
    acc = jnp.zeros((BT, ODIM), jnp.float32)
    for e in range(E):
        he = jnp.dot(qb, w1_ref[e], preferred_element_type=jnp.float32)
        he = jnp.maximum(he + b1_ref[e:e + 1, :], 0.0)
        ye = jnp.dot(he.astype(bf), w2_ref[e],
                     preferred_element_type=jnp.float32)
        ye = ye + b2_ref[e:e + 1, :]
        acc = acc + p[:, e:e + 1] * ye
    part = jnp.dot(acc.astype(bf), wo_ref[...],
                   preferred_element_type=jnp.float32)

    rows = pl.ds(i * BT, BT)
    prev = jnp.where(h_id == 0, 0.0, acc_ref[rows, :])
    total = prev + part
    acc_ref[rows, :] = total
    out_ref[...] = total + bo_ref[...]

    stat_ref[0:1, 0:E] = stat_ref[0:1, 0:E] + psum
    stat_ref[1:2, 0:E] = stat_ref[1:2, 0:E] + csum

    @pl.when(jnp.logical_and(h_id == H - 1, i == nb - 1))
    def _finish():
        ps = stat_ref[0:1, 0:E]
        cn = stat_ref[1:2, 0:E]
        s = jnp.sum(ps * cn, axis=-1, keepdims=True)
        loss_ref[...] = (E / (ntok * ntok)) * s


def kernel(x, Wq, bq, Wk, bk, Wg, bg, W1, b1, W2, b2, Wo, bo):
    B, D = x.shape
    H = Wq.shape[1] // D
    E = Wg.shape[1]
    HID = W1.shape[2]
    ODIM = W2.shape[2]
    NT = Wo.shape[1]
    BT = 256 if B % 256 == 0 else B
    nb = B // BT
    bf = jnp.bfloat16

    body = functools.partial(_moe_body, BT=BT, D=D, H=H, E=E, HID=HID,
                             ODIM=ODIM, NT=NT, nb=nb, ntok=float(B * H))

    const = lambda *shape: pl.BlockSpec(shape, lambda h, i: (0,) * len(shape))
    out0, loss = pl.pallas_call(
        body,
        grid=(H, nb),
        in_specs=[
            pl.BlockSpec((BT, D), lambda h, i: (i, 0)),
            pl.BlockSpec((D, D), lambda h, i: (0, h)),
            pl.BlockSpec((1, 1, D), lambda h, i: (h, 0, 0)),
            pl.BlockSpec((D, D), lambda h, i: (0, h)),
            pl.BlockSpec((1, 1, D), lambda h, i: (h, 0, 0)),
            const(D, E),
            const(1, E),
            const(E, D, HID),
            const(E, HID),
            const(E, HID, ODIM),
            const(E, ODIM),
            pl.BlockSpec((ODIM, NT), lambda h, i: (h, 0)),
            const(1, NT),
        ],
        out_specs=[
            pl.BlockSpec((BT, NT), lambda h, i: (i, 0)),
            pl.BlockSpec((1, 1), lambda h, i: (0, 0)),
        ],
        out_shape=[
            jax.ShapeDtypeStruct((B, NT), jnp.float32),
            jax.ShapeDtypeStruct((1, 1), jnp.float32),
        ],
        scratch_shapes=[
            pltpu.VMEM((B, NT), jnp.float32),
            pltpu.VMEM((8, 128), jnp.float32),
        ],
    )(
        x.astype(bf), Wq.astype(bf), bq.reshape(H, 1, D), Wk.astype(bf),
        bk.reshape(H, 1, D), Wg.astype(bf), bg.reshape(1, E), W1.astype(bf),
        b1, W2.astype(bf), b2, Wo.astype(bf), bo.reshape(1, NT),
    )
    return out0, loss[0, 0]


# speedup vs baseline: 1.5520x; 1.0001x over previous
import functools

import jax
import jax.numpy as jnp
from jax.experimental import pallas as pl
from jax.experimental.pallas import tpu as pltpu


def _moe_body(x_ref, wq_ref, bq_ref, wk_ref, bk_ref, wg_ref, bg_ref,
              w1_ref, b1_ref, w2_ref, b2_ref, wo_ref, bo_ref,
              out_ref, loss_ref, acc_ref, stat_ref, wkwg_ref,
              *, BT, D, H, E, HID, ODIM, NT, nb, ntok):
    h_id = pl.program_id(0)
    i = pl.program_id(1)
    bf = jnp.bfloat16

    @pl.when(jnp.logical_and(h_id == 0, i == 0))
    def _init_stats():
        stat_ref[...] = jnp.zeros_like(stat_ref)

    @pl.when(i == 0)
    def _compose_gate():
        wkwg_ref[...] = jnp.dot(
            wk_ref[...], wg_ref[...],
            preferred_element_type=jnp.float32).astype(bf)

    xb = x_ref[...]
    q = jnp.dot(xb, wq_ref[...], preferred_element_type=jnp.float32)
    q = q + bq_ref[0]

    blog = jnp.dot(bk_ref[0].astype(bf), wg_ref[...],
                   preferred_element_type=jnp.float32)
    logits = jnp.dot(xb, wkwg_ref[...],
                     preferred_element_type=jnp.float32) + blog + bg_ref[...]
    mx = jnp.max(logits, axis=-1, keepdims=True)
    ex = jnp.exp(logits - mx)
    p = ex / jnp.sum(ex, axis=-1, keepdims=True)
    psum = jnp.sum(p, axis=0, keepdims=True)
    eq = logits >= mx
    lane = jax.lax.broadcasted_iota(jnp.int32, (BT, E), 1)
    am = jnp.min(jnp.where(eq, lane, E), axis=-1, keepdims=True)
    oh = (lane == am).astype(jnp.float32)
    csum = jnp.sum(oh, axis=0, keepdims=True)

    qb = q.astype(bf)
    he = jnp.dot(qb, w1_ref[...], preferred_element_type=jnp.float32)
    he = jnp.maximum(he + b1_ref[...], 0.0).astype(bf)
    acc = jnp.zeros((BT, ODIM), jnp.float32)
    for e in range(E):
        ye = jnp.dot(he[:, e * HID:(e + 1) * HID], w2_ref[e],
                     preferred_element_type=jnp.float32)
        ye = ye + b2_ref[e:e + 1, :]
        acc = acc + p[:, e:e + 1] * ye
    part = jnp.dot(acc.astype(bf), wo_ref[...],
                   preferred_element_type=jnp.float32)

    rows = pl.ds(i * BT, BT)
    prev = jnp.where(h_id == 0, 0.0, acc_ref[rows, :])
    total = prev + part
    acc_ref[rows, :] = total
    out_ref[...] = total + bo_ref[...]

    stat_ref[0:1, 0:E] = stat_ref[0:1, 0:E] + psum
    stat_ref[1:2, 0:E] = stat_ref[1:2, 0:E] + csum

    @pl.when(jnp.logical_and(h_id == H - 1, i == nb - 1))
    def _finish():
        ps = stat_ref[0:1, 0:E]
        cn = stat_ref[1:2, 0:E]
        s = jnp.sum(ps * cn, axis=-1, keepdims=True)
        loss_ref[...] = (E / (ntok * ntok)) * s


def kernel(x, Wq, bq, Wk, bk, Wg, bg, W1, b1, W2, b2, Wo, bo):
    B, D = x.shape
    H = Wq.shape[1] // D
    E = Wg.shape[1]
    HID = W1.shape[2]
    ODIM = W2.shape[2]
    NT = Wo.shape[1]
    BT = 256 if B % 256 == 0 else B
    nb = B // BT
    bf = jnp.bfloat16

    body = functools.partial(_moe_body, BT=BT, D=D, H=H, E=E, HID=HID,
                             ODIM=ODIM, NT=NT, nb=nb, ntok=float(B * H))

    W1cat = W1.transpose(1, 0, 2).reshape(D, E * HID).astype(bf)
    b1cat = b1.reshape(1, E * HID)

    const = lambda *shape: pl.BlockSpec(shape, lambda h, i: (0,) * len(shape))
    out0, loss = pl.pallas_call(
        body,
        grid=(H, nb),
        in_specs=[
            pl.BlockSpec((BT, D), lambda h, i: (i, 0)),
            pl.BlockSpec((D, D), lambda h, i: (0, h)),
            pl.BlockSpec((1, 1, D), lambda h, i: (h, 0, 0)),
            pl.BlockSpec((D, D), lambda h, i: (0, h)),
            pl.BlockSpec((1, 1, D), lambda h, i: (h, 0, 0)),
            const(D, E),
            const(1, E),
            const(D, E * HID),
            const(1, E * HID),
            const(E, HID, ODIM),
            const(E, ODIM),
            pl.BlockSpec((ODIM, NT), lambda h, i: (h, 0)),
            const(1, NT),
        ],
        out_specs=[
            pl.BlockSpec((BT, NT), lambda h, i: (i, 0)),
            pl.BlockSpec((1, 1), lambda h, i: (0, 0)),
        ],
        out_shape=[
            jax.ShapeDtypeStruct((B, NT), jnp.float32),
            jax.ShapeDtypeStruct((1, 1), jnp.float32),
        ],
        scratch_shapes=[
            pltpu.VMEM((B, NT), jnp.float32),
            pltpu.VMEM((8, 128), jnp.float32),
            pltpu.VMEM((D, E), jnp.bfloat16),
        ],
    )(
        x.astype(bf), Wq.astype(bf), bq.reshape(H, 1, D), Wk.astype(bf),
        bk.reshape(H, 1, D), Wg.astype(bf), bg.reshape(1, E), W1cat,
        b1cat, W2.astype(bf), b2, Wo.astype(bf), bo.reshape(1, NT),
    )
    return out0, loss[0, 0]


# speedup vs baseline: 1.5534x; 1.0009x over previous
import functools

import jax
import jax.numpy as jnp
from jax.experimental import pallas as pl
from jax.experimental.pallas import tpu as pltpu


def _moe_body(x_ref, wq_ref, bq_ref, wk_ref, bk_ref, wg_ref, bg_ref,
              w1_ref, b1_ref, w2_ref, b2_ref, wo_ref, bo_ref,
              out_ref, loss_ref, acc_ref, stat_ref, wkwg_ref,
              *, BT, D, H, E, HID, ODIM, NT, nb, ntok):
    h_id = pl.program_id(0)
    i = pl.program_id(1)
    bf = jnp.bfloat16

    @pl.when(jnp.logical_and(h_id == 0, i == 0))
    def _init_stats():
        stat_ref[...] = jnp.zeros_like(stat_ref)

    @pl.when(i == 0)
    def _compose_gate():
        wkwg_ref[...] = jnp.dot(
            wk_ref[...], wg_ref[...],
            preferred_element_type=jnp.float32).astype(bf)

    xb = x_ref[...]
    q = jnp.dot(xb, wq_ref[...], preferred_element_type=jnp.float32)
    qb = (q + bq_ref[0]).astype(bf)

    blog = jnp.dot(bk_ref[0].astype(bf), wg_ref[...],
                   preferred_element_type=jnp.float32)
    logits = jnp.dot(xb, wkwg_ref[...],
                     preferred_element_type=jnp.float32) + blog + bg_ref[...]
    mx = jnp.max(logits, axis=-1, keepdims=True)
    ex = jnp.exp(logits - mx)
    p = ex / jnp.sum(ex, axis=-1, keepdims=True)
    psum = jnp.sum(p, axis=0, keepdims=True)
    eq = logits >= mx
    lane = jax.lax.broadcasted_iota(jnp.int32, (BT, E), 1)
    am = jnp.min(jnp.where(eq, lane, E), axis=-1, keepdims=True)
    oh = (lane == am).astype(jnp.float32)
    csum = jnp.sum(oh, axis=0, keepdims=True)

    he = jnp.dot(qb, w1_ref[...], preferred_element_type=jnp.float32)
    he = jnp.maximum(he + b1_ref[...], 0.0).astype(bf)
    acc = jnp.zeros((BT, ODIM), jnp.float32)
    for e in range(E):
        ye = jnp.dot(he[:, e * HID:(e + 1) * HID], w2_ref[e],
                     preferred_element_type=jnp.float32)
        ye = ye + b2_ref[e:e + 1, :]
        acc = acc + p[:, e:e + 1] * ye
    part = jnp.dot(acc.astype(bf), wo_ref[...],
                   preferred_element_type=jnp.float32)

    rows = pl.ds(i * BT, BT)
    prev = jnp.where(h_id == 0, 0.0, acc_ref[rows, :])
    total = prev + part
    acc_ref[rows, :] = total
    out_ref[...] = total + bo_ref[...]

    stat_ref[0:1, 0:E] = stat_ref[0:1, 0:E] + psum
    stat_ref[1:2, 0:E] = stat_ref[1:2, 0:E] + csum

    @pl.when(jnp.logical_and(h_id == H - 1, i == nb - 1))
    def _finish():
        ps = stat_ref[0:1, 0:E]
        cn = stat_ref[1:2, 0:E]
        s = jnp.sum(ps * cn, axis=-1, keepdims=True)
        loss_ref[...] = (E / (ntok * ntok)) * s


def kernel(x, Wq, bq, Wk, bk, Wg, bg, W1, b1, W2, b2, Wo, bo):
    B, D = x.shape
    H = Wq.shape[1] // D
    E = Wg.shape[1]
    HID = W1.shape[2]
    ODIM = W2.shape[2]
    NT = Wo.shape[1]
    BT = 256 if B % 256 == 0 else B
    nb = B // BT
    bf = jnp.bfloat16

    body = functools.partial(_moe_body, BT=BT, D=D, H=H, E=E, HID=HID,
                             ODIM=ODIM, NT=NT, nb=nb, ntok=float(B * H))

    W1cat = W1.transpose(1, 0, 2).reshape(D, E * HID).astype(bf)
    b1cat = b1.reshape(1, E * HID)

    const = lambda *shape: pl.BlockSpec(shape, lambda h, i: (0,) * len(shape))
    out0, loss = pl.pallas_call(
        body,
        grid=(H, nb),
        in_specs=[
            pl.BlockSpec((BT, D), lambda h, i: (i, 0)),
            pl.BlockSpec((D, D), lambda h, i: (0, h)),
            pl.BlockSpec((1, 1, D), lambda h, i: (h, 0, 0)),
            pl.BlockSpec((D, D), lambda h, i: (0, h)),
            pl.BlockSpec((1, 1, D), lambda h, i: (h, 0, 0)),
            const(D, E),
            const(1, E),
            const(D, E * HID),
            const(1, E * HID),
            const(E, HID, ODIM),
            const(E, ODIM),
            pl.BlockSpec((ODIM, NT), lambda h, i: (h, 0)),
            const(1, NT),
        ],
        out_specs=[
            pl.BlockSpec((BT, NT), lambda h, i: (i, 0)),
            pl.BlockSpec((1, 1), lambda h, i: (0, 0)),
        ],
        out_shape=[
            jax.ShapeDtypeStruct((B, NT), jnp.float32),
            jax.ShapeDtypeStruct((1, 1), jnp.float32),
        ],
        scratch_shapes=[
            pltpu.VMEM((B, NT), jnp.float32),
            pltpu.VMEM((8, 128), jnp.float32),
            pltpu.VMEM((D, E), jnp.bfloat16),
        ],
    )(
        x.astype(bf), Wq.astype(bf), bq.reshape(H, 1, D), Wk.astype(bf),
        bk.reshape(H, 1, D), Wg.astype(bf), bg.reshape(1, E), W1cat,
        b1cat, W2.astype(bf), b2, Wo.astype(bf), bo.reshape(1, NT),
    )
    return out0, loss[0, 0]


# speedup vs baseline: 1.7033x; 1.0965x over previous
import functools

import jax
import jax.numpy as jnp
from jax.experimental import pallas as pl
from jax.experimental.pallas import tpu as pltpu


def _moe_body(x_ref, wq_ref, bq_ref, wk_ref, bk_ref, wg_ref, bg_ref,
              w1_ref, b1_ref, w2_ref, b2_ref, wo_ref, bo_ref,
              out_ref, loss_ref, acc_ref, stat_ref, wkwg_ref,
              *, BT, D, H, E, HID, ODIM, NT, nb, ntok):
    h_id = pl.program_id(0)
    i = pl.program_id(1)
    bf = jnp.bfloat16

    @pl.when(jnp.logical_and(h_id == 0, i == 0))
    def _init_stats():
        stat_ref[...] = jnp.zeros_like(stat_ref)

    @pl.when(i == 0)
    def _compose_gate():
        wkwg_ref[...] = jnp.dot(
            wk_ref[...], wg_ref[...],
            preferred_element_type=jnp.float32).astype(bf)

    xb = x_ref[...]
    q = jnp.dot(xb, wq_ref[...], preferred_element_type=jnp.float32)
    qb = (q + bq_ref[0]).astype(bf)

    blog = jnp.dot(bk_ref[0].astype(bf), wg_ref[...],
                   preferred_element_type=jnp.float32)
    logits = jnp.dot(xb, wkwg_ref[...],
                     preferred_element_type=jnp.float32) + blog + bg_ref[...]
    mx = jnp.max(logits, axis=-1, keepdims=True)
    ex = jnp.exp(logits - mx)
    p = ex / jnp.sum(ex, axis=-1, keepdims=True)
    psum = jnp.sum(p, axis=0, keepdims=True)
    eq = logits >= mx
    lane = jax.lax.broadcasted_iota(jnp.int32, (BT, E), 1)
    am = jnp.min(jnp.where(eq, lane, E), axis=-1, keepdims=True)
    oh = (lane == am).astype(jnp.float32)
    csum = jnp.sum(oh, axis=0, keepdims=True)

    acc = jnp.zeros((BT, ODIM), jnp.float32)
    for e in range(E):
        he = jnp.dot(qb, w1_ref[e], preferred_element_type=jnp.float32)
        he = jnp.maximum(he + b1_ref[e:e + 1, :], 0.0).astype(bf)
        ye = jnp.dot(he, w2_ref[e], preferred_element_type=jnp.float32)
        ye = ye + b2_ref[e:e + 1, :]
        acc = acc + p[:, e:e + 1] * ye
    part = jnp.dot(acc.astype(bf), wo_ref[...],
                   preferred_element_type=jnp.float32)

    rows = pl.ds(i * BT, BT)
    prev = jnp.where(h_id == 0, 0.0, acc_ref[rows, :])
    total = prev + part
    acc_ref[rows, :] = total
    out_ref[...] = total + bo_ref[...]

    stat_ref[0:1, 0:E] = stat_ref[0:1, 0:E] + psum
    stat_ref[1:2, 0:E] = stat_ref[1:2, 0:E] + csum

    @pl.when(jnp.logical_and(h_id == H - 1, i == nb - 1))
    def _finish():
        ps = stat_ref[0:1, 0:E]
        cn = stat_ref[1:2, 0:E]
        s = jnp.sum(ps * cn, axis=-1, keepdims=True)
        loss_ref[...] = (E / (ntok * ntok)) * s


def kernel(x, Wq, bq, Wk, bk, Wg, bg, W1, b1, W2, b2, Wo, bo):
    B, D = x.shape
    H = Wq.shape[1] // D
    E = Wg.shape[1]
    HID = W1.shape[2]
    ODIM = W2.shape[2]
    NT = Wo.shape[1]
    BT = 512 if B % 512 == 0 else B
    nb = B // BT
    bf = jnp.bfloat16

    body = functools.partial(_moe_body, BT=BT, D=D, H=H, E=E, HID=HID,
                             ODIM=ODIM, NT=NT, nb=nb, ntok=float(B * H))

    const = lambda *shape: pl.BlockSpec(shape, lambda h, i: (0,) * len(shape))
    out0, loss = pl.pallas_call(
        body,
        grid=(H, nb),
        in_specs=[
            pl.BlockSpec((BT, D), lambda h, i: (i, 0)),
            pl.BlockSpec((D, D), lambda h, i: (0, h)),
            pl.BlockSpec((1, 1, D), lambda h, i: (h, 0, 0)),
            pl.BlockSpec((D, D), lambda h, i: (0, h)),
            pl.BlockSpec((1, 1, D), lambda h, i: (h, 0, 0)),
            const(D, E),
            const(1, E),
            const(E, D, HID),
            const(E, HID),
            const(E, HID, ODIM),
            const(E, ODIM),
            pl.BlockSpec((ODIM, NT), lambda h, i: (h, 0)),
            const(1, NT),
        ],
        out_specs=[
            pl.BlockSpec((BT, NT), lambda h, i: (i, 0)),
            pl.BlockSpec((1, 1), lambda h, i: (0, 0)),
        ],
        out_shape=[
            jax.ShapeDtypeStruct((B, NT), jnp.float32),
            jax.ShapeDtypeStruct((1, 1), jnp.float32),
        ],
        scratch_shapes=[
            pltpu.VMEM((B, NT), jnp.float32),
            pltpu.VMEM((8, 128), jnp.float32),
            pltpu.VMEM((D, E), jnp.bfloat16),
        ],
    )(
        x.astype(bf), Wq.astype(bf), bq.reshape(H, 1, D), Wk.astype(bf),
        bk.reshape(H, 1, D), Wg.astype(bf), bg.reshape(1, E), W1.astype(bf),
        b1, W2.astype(bf), b2, Wo.astype(bf), bo.reshape(1, NT),
    )
    return out0, loss[0, 0]


# speedup vs baseline: 1.7193x; 1.0094x over previous
import functools

import jax
import jax.numpy as jnp
from jax.experimental import pallas as pl
from jax.experimental.pallas import tpu as pltpu


def _cast_body(x_ref, wq_ref, wk_ref, w1_ref, w2_ref, wo_ref,
               xo_ref, wqo_ref, wko_ref, w1o_ref, w2o_ref, woo_ref):
    bf = jnp.bfloat16
    xo_ref[...] = x_ref[...].astype(bf)
    wqo_ref[...] = wq_ref[...].astype(bf)
    wko_ref[...] = wk_ref[...].astype(bf)
    w1o_ref[...] = w1_ref[...].astype(bf)
    w2o_ref[...] = w2_ref[...].astype(bf)
    woo_ref[...] = wo_ref[...].astype(bf)


def _moe_body(x_ref, wq_ref, bq_ref, wk_ref, bk_ref, wg_ref, bg_ref,
              w1_ref, b1_ref, w2_ref, b2_ref, wo_ref, bo_ref,
              out_ref, loss_ref, acc_ref, stat_ref, wkwg_ref,
              *, BT, D, H, E, HID, ODIM, NT, nb, ntok):
    h_id = pl.program_id(0)
    i = pl.program_id(1)
    bf = jnp.bfloat16

    @pl.when(jnp.logical_and(h_id == 0, i == 0))
    def _init_stats():
        stat_ref[...] = jnp.zeros_like(stat_ref)

    @pl.when(i == 0)
    def _compose_gate():
        wkwg_ref[...] = jnp.dot(
            wk_ref[...], wg_ref[...],
            preferred_element_type=jnp.float32).astype(bf)

    xb = x_ref[...]
    q = jnp.dot(xb, wq_ref[...], preferred_element_type=jnp.float32)
    qb = (q + bq_ref[0]).astype(bf)

    blog = jnp.dot(bk_ref[0].astype(bf), wg_ref[...],
                   preferred_element_type=jnp.float32)
    logits = jnp.dot(xb, wkwg_ref[...],
                     preferred_element_type=jnp.float32) + blog + bg_ref[...]
    mx = jnp.max(logits, axis=-1, keepdims=True)
    ex = jnp.exp(logits - mx)
    p = ex / jnp.sum(ex, axis=-1, keepdims=True)
    psum = jnp.sum(p, axis=0, keepdims=True)
    eq = logits >= mx
    lane = jax.lax.broadcasted_iota(jnp.int32, (BT, E), 1)
    am = jnp.min(jnp.where(eq, lane, E), axis=-1, keepdims=True)
    oh = (lane == am).astype(jnp.float32)
    csum = jnp.sum(oh, axis=0, keepdims=True)

    acc = jnp.zeros((BT, ODIM), jnp.float32)
    for e in range(E):
        he = jnp.dot(qb, w1_ref[e], preferred_element_type=jnp.float32)
        he = jnp.maximum(he + b1_ref[e:e + 1, :], 0.0).astype(bf)
        ye = jnp.dot(he, w2_ref[e], preferred_element_type=jnp.float32)
        ye = ye + b2_ref[e:e + 1, :]
        acc = acc + p[:, e:e + 1] * ye
    part = jnp.dot(acc.astype(bf), wo_ref[...],
                   preferred_element_type=jnp.float32)

    rows = pl.ds(i * BT, BT)
    prev = jnp.where(h_id == 0, 0.0, acc_ref[rows, :])
    total = prev + part
    acc_ref[rows, :] = total
    out_ref[...] = total + bo_ref[...]

    stat_ref[0:1, 0:E] = stat_ref[0:1, 0:E] + psum
    stat_ref[1:2, 0:E] = stat_ref[1:2, 0:E] + csum

    @pl.when(jnp.logical_and(h_id == H - 1, i == nb - 1))
    def _finish():
        ps = stat_ref[0:1, 0:E]
        cn = stat_ref[1:2, 0:E]
        s = jnp.sum(ps * cn, axis=-1, keepdims=True)
        loss_ref[...] = (E / (ntok * ntok)) * s


def kernel(x, Wq, bq, Wk, bk, Wg, bg, W1, b1, W2, b2, Wo, bo):
    B, D = x.shape
    H = Wq.shape[1] // D
    E = Wg.shape[1]
    HID = W1.shape[2]
    ODIM = W2.shape[2]
    NT = Wo.shape[1]
    BT = 512 if B % 512 == 0 else B
    nb = B // BT
    bf = jnp.bfloat16

    body = functools.partial(_moe_body, BT=BT, D=D, H=H, E=E, HID=HID,
                             ODIM=ODIM, NT=NT, nb=nb, ntok=float(B * H))

    NC = 8
    xb_, Wqb, Wkb, W1b, W2b, Wob = pl.pallas_call(
        _cast_body,
        grid=(NC,),
        in_specs=[
            pl.BlockSpec((B // NC, D), lambda i: (i, 0)),
            pl.BlockSpec((D, H * D // NC), lambda i: (0, i)),
            pl.BlockSpec((D, H * D // NC), lambda i: (0, i)),
            pl.BlockSpec((1, D, HID), lambda i: (i, 0, 0)),
            pl.BlockSpec((1, HID, ODIM), lambda i: (i, 0, 0)),
            pl.BlockSpec((D // NC, NT), lambda i: (i, 0)),
        ],
        out_specs=[
            pl.BlockSpec((B // NC, D), lambda i: (i, 0)),
            pl.BlockSpec((D, H * D // NC), lambda i: (0, i)),
            pl.BlockSpec((D, H * D // NC), lambda i: (0, i)),
            pl.BlockSpec((1, D, HID), lambda i: (i, 0, 0)),
            pl.BlockSpec((1, HID, ODIM), lambda i: (i, 0, 0)),
            pl.BlockSpec((D // NC, NT), lambda i: (i, 0)),
        ],
        out_shape=[
            jax.ShapeDtypeStruct((B, D), bf),
            jax.ShapeDtypeStruct((D, H * D), bf),
            jax.ShapeDtypeStruct((D, H * D), bf),
            jax.ShapeDtypeStruct((E, D, HID), bf),
            jax.ShapeDtypeStruct((E, HID, ODIM), bf),
            jax.ShapeDtypeStruct((D, NT), bf),
        ],
    )(x, Wq, Wk, W1, W2, Wo)

    const = lambda *shape: pl.BlockSpec(shape, lambda h, i: (0,) * len(shape))
    out0, loss = pl.pallas_call(
        body,
        grid=(H, nb),
        in_specs=[
            pl.BlockSpec((BT, D), lambda h, i: (i, 0)),
            pl.BlockSpec((D, D), lambda h, i: (0, h)),
            pl.BlockSpec((1, 1, D), lambda h, i: (h, 0, 0)),
            pl.BlockSpec((D, D), lambda h, i: (0, h)),
            pl.BlockSpec((1, 1, D), lambda h, i: (h, 0, 0)),
            const(D, E),
            const(1, E),
            const(E, D, HID),
            const(E, HID),
            const(E, HID, ODIM),
            const(E, ODIM),
            pl.BlockSpec((ODIM, NT), lambda h, i: (h, 0)),
            const(1, NT),
        ],
        out_specs=[
            pl.BlockSpec((BT, NT), lambda h, i: (i, 0)),
            pl.BlockSpec((1, 1), lambda h, i: (0, 0)),
        ],
        out_shape=[
            jax.ShapeDtypeStruct((B, NT), jnp.float32),
            jax.ShapeDtypeStruct((1, 1), jnp.float32),
        ],
        scratch_shapes=[
            pltpu.VMEM((B, NT), jnp.float32),
            pltpu.VMEM((8, 128), jnp.float32),
            pltpu.VMEM((D, E), jnp.bfloat16),
        ],
    )(
        xb_, Wqb, bq.reshape(H, 1, D), Wkb,
        bk.reshape(H, 1, D), Wg.astype(bf), bg.reshape(1, E), W1b,
        b1, W2b, b2, Wob, bo.reshape(1, NT),
    )
    return out0, loss[0, 0]


# speedup vs baseline: 1.7560x; 1.0214x over previous
import functools

import jax
import jax.numpy as jnp
from jax.experimental import pallas as pl
from jax.experimental.pallas import tpu as pltpu


def _cast_body(x_ref, wq_ref, wk_ref, w1_ref, w2_ref, wo_ref,
               xo_ref, wqo_ref, wko_ref, w1o_ref, w2o_ref, woo_ref):
    bf = jnp.bfloat16
    xo_ref[...] = x_ref[...].astype(bf)
    wqo_ref[...] = wq_ref[...].astype(bf)
    wko_ref[...] = wk_ref[...].astype(bf)
    w1o_ref[...] = w1_ref[...].astype(bf)
    w2o_ref[...] = w2_ref[...].astype(bf)
    woo_ref[...] = wo_ref[...].astype(bf)


def _moe_body(x_ref, wq_ref, bq_ref, wk_ref, bk_ref, wg_ref, bg_ref,
              w1_ref, b1_ref, w2_ref, b2_ref, wo_ref, bo_ref,
              out_ref, loss_ref, acc_ref, stat_ref, wkwg_ref,
              *, BT, D, H, E, HID, ODIM, NT, nb, ntok):
    h_id = pl.program_id(0)
    i = pl.program_id(1)
    bf = jnp.bfloat16

    @pl.when(jnp.logical_and(h_id == 0, i == 0))
    def _init_stats():
        stat_ref[...] = jnp.zeros_like(stat_ref)

    @pl.when(i == 0)
    def _compose_gate():
        wkwg_ref[...] = jnp.dot(
            wk_ref[...], wg_ref[...],
            preferred_element_type=jnp.float32).astype(bf)

    xb = x_ref[pl.ds(i * BT, BT), :]
    q = jnp.dot(xb, wq_ref[...], preferred_element_type=jnp.float32)
    qb = (q + bq_ref[0]).astype(bf)

    blog = jnp.dot(bk_ref[0].astype(bf), wg_ref[...],
                   preferred_element_type=jnp.float32)
    logits = jnp.dot(xb, wkwg_ref[...],
                     preferred_element_type=jnp.float32) + blog + bg_ref[...]
    mx = jnp.max(logits, axis=-1, keepdims=True)
    ex = jnp.exp(logits - mx)
    p = ex / jnp.sum(ex, axis=-1, keepdims=True)
    psum = jnp.sum(p, axis=0, keepdims=True)
    eq = logits >= mx
    lane = jax.lax.broadcasted_iota(jnp.int32, (BT, E), 1)
    am = jnp.min(jnp.where(eq, lane, E), axis=-1, keepdims=True)
    oh = (lane == am).astype(jnp.float32)
    csum = jnp.sum(oh, axis=0, keepdims=True)

    acc = jnp.zeros((BT, ODIM), jnp.float32)
    for e in range(E):
        he = jnp.dot(qb, w1_ref[e], preferred_element_type=jnp.float32)
        he = jnp.maximum(he + b1_ref[e:e + 1, :], 0.0).astype(bf)
        ye = jnp.dot(he, w2_ref[e], preferred_element_type=jnp.float32)
        ye = ye + b2_ref[e:e + 1, :]
        acc = acc + p[:, e:e + 1] * ye
    part = jnp.dot(acc.astype(bf), wo_ref[...],
                   preferred_element_type=jnp.float32)

    rows = pl.ds(i * BT, BT)
    prev = jnp.where(h_id == 0, 0.0, acc_ref[rows, :])
    total = prev + part
    acc_ref[rows, :] = total
    out_ref[...] = total + bo_ref[...]

    stat_ref[0:1, 0:E] = stat_ref[0:1, 0:E] + psum
    stat_ref[1:2, 0:E] = stat_ref[1:2, 0:E] + csum

    @pl.when(jnp.logical_and(h_id == H - 1, i == nb - 1))
    def _finish():
        ps = stat_ref[0:1, 0:E]
        cn = stat_ref[1:2, 0:E]
        s = jnp.sum(ps * cn, axis=-1, keepdims=True)
        loss_ref[...] = (E / (ntok * ntok)) * s


def kernel(x, Wq, bq, Wk, bk, Wg, bg, W1, b1, W2, b2, Wo, bo):
    B, D = x.shape
    H = Wq.shape[1] // D
    E = Wg.shape[1]
    HID = W1.shape[2]
    ODIM = W2.shape[2]
    NT = Wo.shape[1]
    BT = 1024 if B % 1024 == 0 else B
    nb = B // BT
    bf = jnp.bfloat16

    body = functools.partial(_moe_body, BT=BT, D=D, H=H, E=E, HID=HID,
                             ODIM=ODIM, NT=NT, nb=nb, ntok=float(B * H))

    NC = 8
    xb_, Wqb, Wkb, W1b, W2b, Wob = pl.pallas_call(
        _cast_body,
        grid=(NC,),
        in_specs=[
            pl.BlockSpec((B // NC, D), lambda i: (i, 0)),
            pl.BlockSpec((D, H * D // NC), lambda i: (0, i)),
            pl.BlockSpec((D, H * D // NC), lambda i: (0, i)),
            pl.BlockSpec((1, D, HID), lambda i: (i, 0, 0)),
            pl.BlockSpec((1, HID, ODIM), lambda i: (i, 0, 0)),
            pl.BlockSpec((D // NC, NT), lambda i: (i, 0)),
        ],
        out_specs=[
            pl.BlockSpec((B // NC, D), lambda i: (i, 0)),
            pl.BlockSpec((D, H * D // NC), lambda i: (0, i)),
            pl.BlockSpec((D, H * D // NC), lambda i: (0, i)),
            pl.BlockSpec((1, D, HID), lambda i: (i, 0, 0)),
            pl.BlockSpec((1, HID, ODIM), lambda i: (i, 0, 0)),
            pl.BlockSpec((D // NC, NT), lambda i: (i, 0)),
        ],
        out_shape=[
            jax.ShapeDtypeStruct((B, D), bf),
            jax.ShapeDtypeStruct((D, H * D), bf),
            jax.ShapeDtypeStruct((D, H * D), bf),
            jax.ShapeDtypeStruct((E, D, HID), bf),
            jax.ShapeDtypeStruct((E, HID, ODIM), bf),
            jax.ShapeDtypeStruct((D, NT), bf),
        ],
    )(x, Wq, Wk, W1, W2, Wo)

    const = lambda *shape: pl.BlockSpec(shape, lambda h, i: (0,) * len(shape))
    out0, loss = pl.pallas_call(
        body,
        grid=(H, nb),
        in_specs=[
            const(B, D),
            pl.BlockSpec((D, D), lambda h, i: (0, h)),
            pl.BlockSpec((1, 1, D), lambda h, i: (h, 0, 0)),
            pl.BlockSpec((D, D), lambda h, i: (0, h)),
            pl.BlockSpec((1, 1, D), lambda h, i: (h, 0, 0)),
            const(D, E),
            const(1, E),
            const(E, D, HID),
            const(E, HID),
            const(E, HID, ODIM),
            const(E, ODIM),
            pl.BlockSpec((ODIM, NT), lambda h, i: (h, 0)),
            const(1, NT),
        ],
        out_specs=[
            pl.BlockSpec((BT, NT), lambda h, i: (i, 0)),
            pl.BlockSpec((1, 1), lambda h, i: (0, 0)),
        ],
        out_shape=[
            jax.ShapeDtypeStruct((B, NT), jnp.float32),
            jax.ShapeDtypeStruct((1, 1), jnp.float32),
        ],
        scratch_shapes=[
            pltpu.VMEM((B, NT), jnp.float32),
            pltpu.VMEM((8, 128), jnp.float32),
            pltpu.VMEM((D, E), jnp.bfloat16),
        ],
    )(
        xb_, Wqb, bq.reshape(H, 1, D), Wkb,
        bk.reshape(H, 1, D), Wg.astype(bf), bg.reshape(1, E), W1b,
        b1, W2b, b2, Wob, bo.reshape(1, NT),
    )
    return out0, loss[0, 0]


# speedup vs baseline: 1.8674x; 1.0634x over previous
import functools

import jax
import jax.numpy as jnp
from jax.experimental import pallas as pl
from jax.experimental.pallas import tpu as pltpu


def _cast_body(x_ref, w1_ref, w2_ref, xo_ref, w1o_ref, w2o_ref):
    bf = jnp.bfloat16
    xo_ref[...] = x_ref[...].astype(bf)
    w1o_ref[...] = w1_ref[...].astype(bf)
    w2o_ref[...] = w2_ref[...].astype(bf)


def _moe_body(x_ref, wq_ref, bq_ref, wk_ref, bk_ref, wg_ref, bg_ref,
              w1_ref, b1_ref, w2_ref, b2_ref, wo_ref, bo_ref,
              out_ref, loss_ref, acc_ref, stat_ref, wkwg_ref, wqb_ref,
              *, BT, D, H, E, HID, ODIM, NT, nb, ntok):
    h_id = pl.program_id(0)
    i = pl.program_id(1)
    bf = jnp.bfloat16

    @pl.when(jnp.logical_and(h_id == 0, i == 0))
    def _init_stats():
        stat_ref[...] = jnp.zeros_like(stat_ref)

    @pl.when(i == 0)
    def _prep_head():
        wqb_ref[...] = wq_ref[...].astype(bf)
        wkwg_ref[...] = jnp.dot(
            wk_ref[...], wg_ref[...],
            preferred_element_type=jnp.float32).astype(bf)

    xb = x_ref[pl.ds(i * BT, BT), :]
    q = jnp.dot(xb, wqb_ref[...], preferred_element_type=jnp.float32)
    qb = (q + bq_ref[0]).astype(bf)

    blog = jnp.dot(bk_ref[0], wg_ref[...],
                   preferred_element_type=jnp.float32)
    logits = jnp.dot(xb, wkwg_ref[...],
                     preferred_element_type=jnp.float32) + blog + bg_ref[...]
    mx = jnp.max(logits, axis=-1, keepdims=True)
    ex = jnp.exp(logits - mx)
    p = ex / jnp.sum(ex, axis=-1, keepdims=True)
    psum = jnp.sum(p, axis=0, keepdims=True)
    eq = logits >= mx
    lane = jax.lax.broadcasted_iota(jnp.int32, (BT, E), 1)
    am = jnp.min(jnp.where(eq, lane, E), axis=-1, keepdims=True)
    oh = (lane == am).astype(jnp.float32)
    csum = jnp.sum(oh, axis=0, keepdims=True)

    acc = jnp.zeros((BT, ODIM), jnp.float32)
    for e in range(E):
        he = jnp.dot(qb, w1_ref[e], preferred_element_type=jnp.float32)
        he = jnp.maximum(he + b1_ref[e:e + 1, :], 0.0).astype(bf)
        ye = jnp.dot(he, w2_ref[e], preferred_element_type=jnp.float32)
        ye = ye + b2_ref[e:e + 1, :]
        acc = acc + p[:, e:e + 1] * ye
    part = jnp.dot(acc.astype(bf), wo_ref[...].astype(bf),
                   preferred_element_type=jnp.float32)

    rows = pl.ds(i * BT, BT)
    prev = jnp.where(h_id == 0, 0.0, acc_ref[rows, :])
    total = prev + part
    acc_ref[rows, :] = total
    out_ref[...] = total + bo_ref[...]

    stat_ref[0:1, 0:E] = stat_ref[0:1, 0:E] + psum
    stat_ref[1:2, 0:E] = stat_ref[1:2, 0:E] + csum

    @pl.when(jnp.logical_and(h_id == H - 1, i == nb - 1))
    def _finish():
        ps = stat_ref[0:1, 0:E]
        cn = stat_ref[1:2, 0:E]
        s = jnp.sum(ps * cn, axis=-1, keepdims=True)
        loss_ref[...] = (E / (ntok * ntok)) * s


def kernel(x, Wq, bq, Wk, bk, Wg, bg, W1, b1, W2, b2, Wo, bo):
    B, D = x.shape
    H = Wq.shape[1] // D
    E = Wg.shape[1]
    HID = W1.shape[2]
    ODIM = W2.shape[2]
    NT = Wo.shape[1]
    BT = 1024 if B % 1024 == 0 else B
    nb = B // BT
    bf = jnp.bfloat16

    body = functools.partial(_moe_body, BT=BT, D=D, H=H, E=E, HID=HID,
                             ODIM=ODIM, NT=NT, nb=nb, ntok=float(B * H))

    NC = 8
    xb_, W1b, W2b = pl.pallas_call(
        _cast_body,
        grid=(NC,),
        in_specs=[
            pl.BlockSpec((B // NC, D), lambda i: (i, 0)),
            pl.BlockSpec((1, D, HID), lambda i: (i, 0, 0)),
            pl.BlockSpec((1, HID, ODIM), lambda i: (i, 0, 0)),
        ],
        out_specs=[
            pl.BlockSpec((B // NC, D), lambda i: (i, 0)),
            pl.BlockSpec((1, D, HID), lambda i: (i, 0, 0)),
            pl.BlockSpec((1, HID, ODIM), lambda i: (i, 0, 0)),
        ],
        out_shape=[
            jax.ShapeDtypeStruct((B, D), bf),
            jax.ShapeDtypeStruct((E, D, HID), bf),
            jax.ShapeDtypeStruct((E, HID, ODIM), bf),
        ],
    )(x, W1, W2)

    const = lambda *shape: pl.BlockSpec(shape, lambda h, i: (0,) * len(shape))
    out0, loss = pl.pallas_call(
        body,
        grid=(H, nb),
        in_specs=[
            const(B, D),
            pl.BlockSpec((D, D), lambda h, i: (0, h)),
            pl.BlockSpec((1, 1, D), lambda h, i: (h, 0, 0)),
            pl.BlockSpec((D, D), lambda h, i: (0, h)),
            pl.BlockSpec((1, 1, D), lambda h, i: (h, 0, 0)),
            const(D, E),
            const(1, E),
            const(E, D, HID),
            const(E, HID),
            const(E, HID, ODIM),
            const(E, ODIM),
            pl.BlockSpec((ODIM, NT), lambda h, i: (h, 0)),
            const(1, NT),
        ],
        out_specs=[
            pl.BlockSpec((BT, NT), lambda h, i: (i, 0)),
            pl.BlockSpec((1, 1), lambda h, i: (0, 0)),
        ],
        out_shape=[
            jax.ShapeDtypeStruct((B, NT), jnp.float32),
            jax.ShapeDtypeStruct((1, 1), jnp.float32),
        ],
        scratch_shapes=[
            pltpu.VMEM((B, NT), jnp.float32),
            pltpu.VMEM((8, 128), jnp.float32),
            pltpu.VMEM((D, E), jnp.bfloat16),
            pltpu.VMEM((D, D), jnp.bfloat16),
        ],
    )(
        xb_, Wq, bq.reshape(H, 1, D), Wk,
        bk.reshape(H, 1, D), Wg, bg.reshape(1, E), W1b,
        b1, W2b, b2, Wo, bo.reshape(1, NT),
    )
    return out0, loss[0, 0]
